# baseline (speedup 1.0000x reference)
import functools

import jax
import jax.numpy as jnp
from jax import lax
from jax.experimental import pallas as pl
from jax.experimental.pallas import tpu as pltpu
from jax.experimental.pallas import tpu_sc as plsc

_N = 10000
_E = 320000
_F = 16
_NTILES = 16
_NW = 32
_EPT = _E // _NW
_C = 100
_M = _EPT // _C
_K = 5
_NB = _M // _K
_RPT = _N // _NTILES


@functools.cache
def _sc_mesh():
    return plsc.VectorSubcoreMesh(
        core_axis_name="c", subcore_axis_name="s", num_cores=2, num_subcores=_NTILES
    )


def _rsqrt_nr(x, iters=3):
    i = lax.bitcast_convert_type(x, jnp.int32)
    y = lax.bitcast_convert_type(jnp.int32(0x5F3759DF) - (i >> 1), jnp.float32)
    for _ in range(iters):
        y = y * (1.5 - 0.5 * x * y * y)
    return y


def _zero_slice(tmp, acc_sh, s):
    def zloop(j, carry):
        tmp[j] = jnp.zeros((_F,), jnp.float32)
        return carry
    lax.fori_loop(0, _RPT, zloop, 0)
    pltpu.sync_copy(tmp, acc_sh.at[pl.ds(s * _RPT, _RPT)])


def _writeback(acc_sh, out_hbm, c, s):
    pltpu.sync_copy(
        acc_sh.at[pl.ds(s * _RPT, _RPT)],
        out_hbm.at[c, pl.ds(s * _RPT, _RPT)],
    )


def _load_idx(src_hbm, dst_hbm, src_idx, dst_idx, wid):
    pltpu.sync_copy(src_hbm.at[pl.ds(wid * _M, _M)], src_idx)
    pltpu.sync_copy(dst_hbm.at[pl.ds(wid * _M, _M)], dst_idx)


def _prop_phase(z_sh, dummy_hbm, acc_sh, src_idx, dst_idx,
                rows_a, rows_b, sga, sgb, ssa, ssb):
    def fire_gathers(j, buf, sem):
        for i in range(_K):
            pltpu.async_copy(
                z_sh.at[src_idx.at[j * _K + i]], buf.at[pl.ds(i * _C, _C)], sem
            )

    def fire_scatters(j, buf, sem):
        for i in range(_K):
            pltpu.async_copy(
                buf.at[pl.ds(i * _C, _C)], acc_sh.at[dst_idx.at[j * _K + i]],
                sem, add=True,
            )

    def drain(sem, buf):
        pltpu.make_async_copy(dummy_hbm.at[pl.ds(0, _K * _C)], buf, sem).wait()

    def body(j, carry):
        even = lax.rem(j, 2) == 0

        @pl.when(even)
        def _():
            @pl.when(j > 1)
            def _():
                drain(ssa, rows_a)
            fire_gathers(j, rows_a, sga)

            @pl.when(j > 0)
            def _():
                drain(sgb, rows_b)
                fire_scatters(j - 1, rows_b, ssb)

        @pl.when(jnp.logical_not(even))
        def _():
            @pl.when(j > 1)
            def _():
                drain(ssb, rows_b)
            fire_gathers(j, rows_b, sgb)
            drain(sga, rows_a)
            fire_scatters(j - 1, rows_a, ssa)

        return carry

    lax.fori_loop(0, _NB, body, 0)
    if _NB % 2 == 1:
        drain(sga, rows_a)
        fire_scatters(_NB - 1, rows_a, ssa)
        drain(ssb, rows_b)
        drain(ssa, rows_a)
    else:
        drain(sgb, rows_b)
        fire_scatters(_NB - 1, rows_b, ssb)
        drain(ssa, rows_a)
        drain(ssb, rows_b)


def _deg_body(dst_hbm, out_hbm, dst_idx, ones_v, drain_v, zbuf, acc_sh, sems):
    c = lax.axis_index("c")
    s = lax.axis_index("s")
    wid = c * _NTILES + s

    def oloop(j, carry):
        ones_v[j] = jnp.ones((_F,), jnp.float32)
        return carry
    lax.fori_loop(0, _C, oloop, 0)
    pltpu.sync_copy(dst_hbm.at[pl.ds(wid * _M, _M)], dst_idx)
    _zero_slice(zbuf, acc_sh, s)
    plsc.subcore_barrier()

    def _drain(sem):
        pltpu.make_async_copy(
            out_hbm.at[0, pl.ds(0, _K * _C)], drain_v, sem
        ).wait()

    def body(j, carry):
        @pl.when(j > 0)
        def _():
            _drain(sems)
        for i in range(_K):
            g = j * _K + i
            pltpu.async_copy(ones_v, acc_sh.at[dst_idx.at[g]], sems, add=True)
        return carry

    lax.fori_loop(0, _NB, body, 0)
    _drain(sems)

    plsc.subcore_barrier()
    _writeback(acc_sh, out_hbm, c, s)


def _p1_body(src_hbm, dst_hbm, degp_hbm, z1_hbm,
             out_hbm, zs_hbm, dinv_hbm,
             src_idx, dst_idx, rows_a, rows_b, dp0b, dp1b, zlocal,
             z_sh, acc_sh, sga, sgb, ssa, ssb, swb):
    c = lax.axis_index("c")
    s = lax.axis_index("s")
    wid = c * _NTILES + s
    r0 = s * _RPT

    _load_idx(src_hbm, dst_hbm, src_idx, dst_idx, wid)
    pltpu.sync_copy(degp_hbm.at[0, pl.ds(r0, _RPT)], dp0b)
    pltpu.sync_copy(degp_hbm.at[1, pl.ds(r0, _RPT)], dp1b)
    pltpu.sync_copy(z1_hbm.at[pl.ds(r0, _RPT)], zlocal)

    def drow(r, carry):
        di = _rsqrt_nr(dp0b[r] + dp1b[r] + 1.0)
        zlocal[r] = zlocal[r] * di
        dp0b[r] = di
        return carry

    lax.fori_loop(0, _RPT, drow, 0)
    pltpu.sync_copy(zlocal, z_sh.at[pl.ds(r0, _RPT)])

    @pl.when(c == 0)
    def _():
        pltpu.async_copy(zlocal, zs_hbm.at[pl.ds(r0, _RPT)], swb)
        pltpu.async_copy(dp0b, dinv_hbm.at[pl.ds(r0, _RPT)], swb)

    _zero_slice(dp1b, acc_sh, s)
    plsc.subcore_barrier()
    _prop_phase(z_sh, z1_hbm, acc_sh, src_idx, dst_idx,
                rows_a, rows_b, sga, sgb, ssa, ssb)

    @pl.when(c == 0)
    def _():
        pltpu.make_async_copy(zlocal, zs_hbm.at[pl.ds(r0, _RPT)], swb).wait()
        pltpu.make_async_copy(dp0b, dinv_hbm.at[pl.ds(r0, _RPT)], swb).wait()

    plsc.subcore_barrier()
    _writeback(acc_sh, out_hbm, c, s)


def _pm_body(with_w, src_hbm, dst_hbm, sp_hbm, zs_hbm, dinv_hbm, par_hbm,
             out_hbm, znext_hbm,
             src_idx, dst_idx, rows_a, rows_b,
             sp0b, sp1b, zsb, dinvb, hb, wbuf, sbuf, stb,
             z_sh, acc_sh, stats_sh, sga, sgb, ssa, ssb):
    c = lax.axis_index("c")
    s = lax.axis_index("s")
    wid = c * _NTILES + s
    r0 = s * _RPT

    _load_idx(src_hbm, dst_hbm, src_idx, dst_idx, wid)
    pltpu.sync_copy(sp_hbm.at[0, pl.ds(r0, _RPT)], sp0b)
    pltpu.sync_copy(sp_hbm.at[1, pl.ds(r0, _RPT)], sp1b)
    pltpu.sync_copy(zs_hbm.at[pl.ds(r0, _RPT)], zsb)
    pltpu.sync_copy(dinv_hbm.at[pl.ds(r0, _RPT)], dinvb)
    pltpu.sync_copy(par_hbm, wbuf)

    bvec = wbuf[0]
    gvec = wbuf[1]
    bevec = wbuf[2]
    zero16 = jnp.zeros((_F,), jnp.float32)

    def arow(r, carry):
        ssum, ssq = carry
        h = dinvb[r] * (sp0b[r] + sp1b[r] + zsb[r]) + bvec
        hb[r] = h
        return (ssum + h, ssq + h * h)

    ssum, ssq = lax.fori_loop(0, _RPT, arow, (zero16, zero16))
    sbuf[0] = ssum
    sbuf[1] = ssq
    pltpu.sync_copy(sbuf, stats_sh.at[s])
    plsc.subcore_barrier()
    pltpu.sync_copy(stats_sh, stb)

    def sred(t, carry):
        return (carry[0] + stb[t, 0], carry[1] + stb[t, 1])

    tsum, tsq = lax.fori_loop(0, _NTILES, sred, (zero16, zero16))
    mean = tsum * (1.0 / _N)
    var = tsq * (1.0 / _N) - mean * mean
    scale = gvec * _rsqrt_nr(var + 1e-5)
    shift = bevec - mean * scale

    if with_w:
        wk = [wbuf[3 + k] for k in range(_F)]

    def brow(r, carry):
        a = jnp.maximum(hb[r] * scale + shift, 0.0)
        if with_w:
            z = zero16
            for k in range(_F):
                z = z + a[k] * wk[k]
        else:
            z = a
        zsb[r] = z * dinvb[r]
        return carry

    lax.fori_loop(0, _RPT, brow, 0)

    pltpu.sync_copy(zsb, z_sh.at[pl.ds(r0, _RPT)])

    @pl.when(c == 0)
    def _():
        pltpu.sync_copy(zsb, znext_hbm.at[pl.ds(r0, _RPT)])

    _zero_slice(sp0b, acc_sh, s)
    plsc.subcore_barrier()
    _prop_phase(z_sh, zs_hbm, acc_sh, src_idx, dst_idx,
                rows_a, rows_b, sga, sgb, ssa, ssb)
    plsc.subcore_barrier()
    _writeback(acc_sh, out_hbm, c, s)


_IDX_T = pltpu.VMEM((_M, _C), jnp.int32)
_ROWS_T = pltpu.VMEM((_K * _C, _F), jnp.float32)
_SLICE_T = pltpu.VMEM((_RPT, _F), jnp.float32)
_SEMS4 = [pltpu.SemaphoreType.DMA] * 4


@functools.cache
def _deg_sc():
    return pl.kernel(
        _deg_body,
        out_type=jax.ShapeDtypeStruct((2, _N, _F), jnp.float32),
        mesh=_sc_mesh(),
        scratch_types=[
            _IDX_T,
            pltpu.VMEM((_C, _F), jnp.float32),
            _ROWS_T,
            _SLICE_T,
            pltpu.VMEM_SHARED((_N, _F), jnp.float32),
            pltpu.SemaphoreType.DMA,
        ],
        compiler_params=pltpu.CompilerParams(use_tc_tiling_on_sc=False),
    )


@functools.cache
def _p1_sc():
    return pl.kernel(
        _p1_body,
        out_type=[
            jax.ShapeDtypeStruct((2, _N, _F), jnp.float32),
            jax.ShapeDtypeStruct((_N, _F), jnp.float32),
            jax.ShapeDtypeStruct((_N, _F), jnp.float32),
        ],
        mesh=_sc_mesh(),
        scratch_types=[
            _IDX_T, _IDX_T, _ROWS_T, _ROWS_T,
            _SLICE_T, _SLICE_T, _SLICE_T,
            pltpu.VMEM_SHARED((_N, _F), jnp.float32),
            pltpu.VMEM_SHARED((_N, _F), jnp.float32),
            *_SEMS4,
            pltpu.SemaphoreType.DMA,
        ],
        compiler_params=pltpu.CompilerParams(use_tc_tiling_on_sc=False),
    )


@functools.cache
def _pm_sc(with_w):
    npar = 3 + (_F if with_w else 0)
    return pl.kernel(
        functools.partial(_pm_body, with_w),
        out_type=[
            jax.ShapeDtypeStruct((2, _N, _F), jnp.float32),
            jax.ShapeDtypeStruct((_N, _F), jnp.float32),
        ],
        mesh=_sc_mesh(),
        scratch_types=[
            _IDX_T, _IDX_T, _ROWS_T, _ROWS_T,
            _SLICE_T, _SLICE_T, _SLICE_T, _SLICE_T, _SLICE_T,
            pltpu.VMEM((npar, _F), jnp.float32),
            pltpu.VMEM((2, _F), jnp.float32),
            pltpu.VMEM((_NTILES, 2, _F), jnp.float32),
            pltpu.VMEM_SHARED((_N, _F), jnp.float32),
            pltpu.VMEM_SHARED((_N, _F), jnp.float32),
            pltpu.VMEM_SHARED((_NTILES, 2, _F), jnp.float32),
            *_SEMS4,
        ],
        compiler_params=pltpu.CompilerParams(use_tc_tiling_on_sc=False),
    )


_R = 400


def _mm1_body(x_ref, w_ref, z1_ref):
    z1_ref[...] = jnp.dot(
        x_ref[...], w_ref[...], preferred_element_type=jnp.float32
    )


def _mm1_tc(x, W1):
    nb = _N // _R
    return pl.pallas_call(
        _mm1_body,
        grid=(nb,),
        in_specs=[
            pl.BlockSpec((_R, 128), lambda i: (i, 0)),
            pl.BlockSpec((128, _F), lambda i: (0, 0)),
        ],
        out_specs=pl.BlockSpec((_R, _F), lambda i: (i, 0)),
        out_shape=jax.ShapeDtypeStruct((_N, _F), jnp.float32),
    )(x, W1)


def _post_body(sp_ref, as_ref, dinv_ref, w_ref, b_ref, out_ref):
    p = dinv_ref[...] * (sp_ref[0] + sp_ref[1] + as_ref[...])
    out_ref[...] = (
        jnp.dot(p, w_ref[...], preferred_element_type=jnp.float32) + b_ref[...]
    )


def _post_tc(sp, a2s, dinv_b, W3, b3):
    nb = _N // _R
    return pl.pallas_call(
        _post_body,
        grid=(nb,),
        in_specs=[
            pl.BlockSpec((2, _R, _F), lambda i: (0, i, 0)),
            pl.BlockSpec((_R, _F), lambda i: (i, 0)),
            pl.BlockSpec((_R, _F), lambda i: (i, 0)),
            pl.BlockSpec((_F, 128), lambda i: (0, 0)),
            pl.BlockSpec((1, 128), lambda i: (0, 0)),
        ],
        out_specs=pl.BlockSpec((_R, 128), lambda i: (i, 0)),
        out_shape=jax.ShapeDtypeStruct((_N, 128), jnp.float32),
    )(sp, a2s, dinv_b, W3, b3.reshape(1, 128))


def kernel(x, edge_index, W1, b1, g1, be1, W2, b2, g2, be2, W3, b3):
    src = edge_index[0].reshape(_E // _C, _C)
    dst = edge_index[1].reshape(_E // _C, _C)
    par1 = jnp.concatenate([b1[None, :], g1[None, :], be1[None, :], W2], axis=0)
    par2 = jnp.concatenate([b2[None, :], g2[None, :], be2[None, :]], axis=0)

    degp = _deg_sc()(dst)
    z1 = _mm1_tc(x, W1)
    s1p, z1s, dinv_b = _p1_sc()(src, dst, degp, z1)
    s2p, z2s = _pm_sc(True)(src, dst, s1p, z1s, dinv_b, par1)
    s3p, a2s = _pm_sc(False)(src, dst, s2p, z2s, dinv_b, par2)
    return _post_tc(s3p, a2s, dinv_b, W3, b3)


# speedup vs baseline: 1.0567x; 1.0567x over previous
import functools

import jax
import jax.numpy as jnp
from jax import lax
from jax.experimental import pallas as pl
from jax.experimental.pallas import tpu as pltpu
from jax.experimental.pallas import tpu_sc as plsc

_N = 10000
_E = 320000
_F = 16
_NTILES = 16
_NW = 32
_EPT = _E // _NW
_C = 80
_M = _EPT // _C
_K = 5
_NB = _M // _K
_RPT = _N // _NTILES


@functools.cache
def _sc_mesh():
    return plsc.VectorSubcoreMesh(
        core_axis_name="c", subcore_axis_name="s", num_cores=2, num_subcores=_NTILES
    )


def _rsqrt_nr(x, iters=3):
    i = lax.bitcast_convert_type(x, jnp.int32)
    y = lax.bitcast_convert_type(jnp.int32(0x5F3759DF) - (i >> 1), jnp.float32)
    for _ in range(iters):
        y = y * (1.5 - 0.5 * x * y * y)
    return y


def _zero_slice(tmp, acc_sh, s):
    def zloop(j, carry):
        tmp[j] = jnp.zeros((_F,), jnp.float32)
        return carry
    lax.fori_loop(0, _RPT, zloop, 0)
    pltpu.sync_copy(tmp, acc_sh.at[pl.ds(s * _RPT, _RPT)])


def _writeback(acc_sh, out_hbm, c, s):
    pltpu.sync_copy(
        acc_sh.at[pl.ds(s * _RPT, _RPT)],
        out_hbm.at[c, pl.ds(s * _RPT, _RPT)],
    )


def _load_idx(src_hbm, dst_hbm, src_idx, dst_idx, wid):
    pltpu.sync_copy(src_hbm.at[pl.ds(wid * _M, _M)], src_idx)
    pltpu.sync_copy(dst_hbm.at[pl.ds(wid * _M, _M)], dst_idx)


def _prop_phase(z_sh, dummy_hbm, acc_sh, src_idx, dst_idx,
                rows_a, rows_b, sga, sgb, ssa, ssb):
    def fire_gathers(j, buf, sem):
        for i in range(_K):
            pltpu.async_copy(
                z_sh.at[src_idx.at[j * _K + i]], buf.at[pl.ds(i * _C, _C)], sem
            )

    def fire_scatters(j, buf, sem):
        for i in range(_K):
            pltpu.async_copy(
                buf.at[pl.ds(i * _C, _C)], acc_sh.at[dst_idx.at[j * _K + i]],
                sem, add=True,
            )

    def drain(sem, buf):
        pltpu.make_async_copy(dummy_hbm.at[pl.ds(0, _K * _C)], buf, sem).wait()

    def body(j, carry):
        even = lax.rem(j, 2) == 0

        @pl.when(even)
        def _():
            @pl.when(j > 1)
            def _():
                drain(ssa, rows_a)
            fire_gathers(j, rows_a, sga)

            @pl.when(j > 0)
            def _():
                drain(sgb, rows_b)
                fire_scatters(j - 1, rows_b, ssb)

        @pl.when(jnp.logical_not(even))
        def _():
            @pl.when(j > 1)
            def _():
                drain(ssb, rows_b)
            fire_gathers(j, rows_b, sgb)
            drain(sga, rows_a)
            fire_scatters(j - 1, rows_a, ssa)

        return carry

    lax.fori_loop(0, _NB, body, 0)
    if _NB % 2 == 1:
        drain(sga, rows_a)
        fire_scatters(_NB - 1, rows_a, ssa)
        drain(ssb, rows_b)
        drain(ssa, rows_a)
    else:
        drain(sgb, rows_b)
        fire_scatters(_NB - 1, rows_b, ssb)
        drain(ssa, rows_a)
        drain(ssb, rows_b)


def _deg_body(dst_hbm, out_hbm, dst_idx, ones_v, drain_v, zbuf, acc_sh, sems):
    c = lax.axis_index("c")
    s = lax.axis_index("s")
    wid = c * _NTILES + s

    def oloop(j, carry):
        ones_v[j] = jnp.ones((_F,), jnp.float32)
        return carry
    lax.fori_loop(0, _C, oloop, 0)
    pltpu.sync_copy(dst_hbm.at[pl.ds(wid * _M, _M)], dst_idx)
    _zero_slice(zbuf, acc_sh, s)
    plsc.subcore_barrier()

    def _drain(sem):
        pltpu.make_async_copy(
            out_hbm.at[0, pl.ds(0, _K * _C)], drain_v, sem
        ).wait()

    def body(j, carry):
        @pl.when(j > 0)
        def _():
            _drain(sems)
        for i in range(_K):
            g = j * _K + i
            pltpu.async_copy(ones_v, acc_sh.at[dst_idx.at[g]], sems, add=True)
        return carry

    lax.fori_loop(0, _NB, body, 0)
    _drain(sems)

    plsc.subcore_barrier()
    _writeback(acc_sh, out_hbm, c, s)


def _p1_body(src_hbm, dst_hbm, degp_hbm, z1_hbm,
             out_hbm, zs_hbm, dinv_hbm,
             src_idx, dst_idx, rows_a, rows_b, dp0b, dp1b, zlocal,
             z_sh, acc_sh, sga, sgb, ssa, ssb, swb):
    c = lax.axis_index("c")
    s = lax.axis_index("s")
    wid = c * _NTILES + s
    r0 = s * _RPT

    _load_idx(src_hbm, dst_hbm, src_idx, dst_idx, wid)
    pltpu.sync_copy(degp_hbm.at[0, pl.ds(r0, _RPT)], dp0b)
    pltpu.sync_copy(degp_hbm.at[1, pl.ds(r0, _RPT)], dp1b)
    pltpu.sync_copy(z1_hbm.at[pl.ds(r0, _RPT)], zlocal)

    def drow(r, carry):
        di = _rsqrt_nr(dp0b[r] + dp1b[r] + 1.0, iters=2)
        zlocal[r] = zlocal[r] * di
        dp0b[r] = di
        return carry

    lax.fori_loop(0, _RPT, drow, 0)
    pltpu.sync_copy(zlocal, z_sh.at[pl.ds(r0, _RPT)])

    @pl.when(c == 0)
    def _():
        pltpu.async_copy(zlocal, zs_hbm.at[pl.ds(r0, _RPT)], swb)
        pltpu.async_copy(dp0b, dinv_hbm.at[pl.ds(r0, _RPT)], swb)

    _zero_slice(dp1b, acc_sh, s)
    plsc.subcore_barrier()
    _prop_phase(z_sh, z1_hbm, acc_sh, src_idx, dst_idx,
                rows_a, rows_b, sga, sgb, ssa, ssb)

    @pl.when(c == 0)
    def _():
        pltpu.make_async_copy(zlocal, zs_hbm.at[pl.ds(r0, _RPT)], swb).wait()
        pltpu.make_async_copy(dp0b, dinv_hbm.at[pl.ds(r0, _RPT)], swb).wait()

    plsc.subcore_barrier()
    _writeback(acc_sh, out_hbm, c, s)


def _pm_body(with_w, src_hbm, dst_hbm, sp_hbm, zs_hbm, dinv_hbm, par_hbm,
             out_hbm, znext_hbm,
             src_idx, dst_idx, rows_a, rows_b,
             sp0b, sp1b, zsb, dinvb, hb, wbuf, sbuf, stb,
             z_sh, acc_sh, stats_sh, sga, sgb, ssa, ssb):
    c = lax.axis_index("c")
    s = lax.axis_index("s")
    wid = c * _NTILES + s
    r0 = s * _RPT

    _load_idx(src_hbm, dst_hbm, src_idx, dst_idx, wid)
    pltpu.sync_copy(sp_hbm.at[0, pl.ds(r0, _RPT)], sp0b)
    pltpu.sync_copy(sp_hbm.at[1, pl.ds(r0, _RPT)], sp1b)
    pltpu.sync_copy(zs_hbm.at[pl.ds(r0, _RPT)], zsb)
    pltpu.sync_copy(dinv_hbm.at[pl.ds(r0, _RPT)], dinvb)
    pltpu.sync_copy(par_hbm, wbuf)

    bvec = wbuf[0]
    gvec = wbuf[1]
    bevec = wbuf[2]
    zero16 = jnp.zeros((_F,), jnp.float32)

    def arow(r, carry):
        ssum, ssq = carry
        h = dinvb[r] * (sp0b[r] + sp1b[r] + zsb[r]) + bvec
        hb[r] = h
        return (ssum + h, ssq + h * h)

    ssum, ssq = lax.fori_loop(0, _RPT, arow, (zero16, zero16))
    sbuf[0] = ssum
    sbuf[1] = ssq
    pltpu.sync_copy(sbuf, stats_sh.at[s])
    plsc.subcore_barrier()
    pltpu.sync_copy(stats_sh, stb)

    def sred(t, carry):
        return (carry[0] + stb[t, 0], carry[1] + stb[t, 1])

    tsum, tsq = lax.fori_loop(0, _NTILES, sred, (zero16, zero16))
    mean = tsum * (1.0 / _N)
    var = tsq * (1.0 / _N) - mean * mean
    scale = gvec * _rsqrt_nr(var + 1e-5)
    shift = bevec - mean * scale

    if with_w:
        wk = [wbuf[3 + k] for k in range(_F)]

    def brow(r, carry):
        a = jnp.maximum(hb[r] * scale + shift, 0.0)
        if with_w:
            z = zero16
            for k in range(_F):
                z = z + a[k] * wk[k]
        else:
            z = a
        zsb[r] = z * dinvb[r]
        return carry

    lax.fori_loop(0, _RPT, brow, 0)

    pltpu.sync_copy(zsb, z_sh.at[pl.ds(r0, _RPT)])

    @pl.when(c == 0)
    def _():
        pltpu.sync_copy(zsb, znext_hbm.at[pl.ds(r0, _RPT)])

    _zero_slice(sp0b, acc_sh, s)
    plsc.subcore_barrier()
    _prop_phase(z_sh, zs_hbm, acc_sh, src_idx, dst_idx,
                rows_a, rows_b, sga, sgb, ssa, ssb)
    plsc.subcore_barrier()
    _writeback(acc_sh, out_hbm, c, s)


_IDX_T = pltpu.VMEM((_M, _C), jnp.int32)
_ROWS_T = pltpu.VMEM((_K * _C, _F), jnp.float32)
_SLICE_T = pltpu.VMEM((_RPT, _F), jnp.float32)
_SEMS4 = [pltpu.SemaphoreType.DMA] * 4


@functools.cache
def _deg_sc():
    return pl.kernel(
        _deg_body,
        out_type=jax.ShapeDtypeStruct((2, _N, _F), jnp.float32),
        mesh=_sc_mesh(),
        scratch_types=[
            _IDX_T,
            pltpu.VMEM((_C, _F), jnp.float32),
            _ROWS_T,
            _SLICE_T,
            pltpu.VMEM_SHARED((_N, _F), jnp.float32),
            pltpu.SemaphoreType.DMA,
        ],
        compiler_params=pltpu.CompilerParams(use_tc_tiling_on_sc=False),
    )


@functools.cache
def _p1_sc():
    return pl.kernel(
        _p1_body,
        out_type=[
            jax.ShapeDtypeStruct((2, _N, _F), jnp.float32),
            jax.ShapeDtypeStruct((_N, _F), jnp.float32),
            jax.ShapeDtypeStruct((_N, _F), jnp.float32),
        ],
        mesh=_sc_mesh(),
        scratch_types=[
            _IDX_T, _IDX_T, _ROWS_T, _ROWS_T,
            _SLICE_T, _SLICE_T, _SLICE_T,
            pltpu.VMEM_SHARED((_N, _F), jnp.float32),
            pltpu.VMEM_SHARED((_N, _F), jnp.float32),
            *_SEMS4,
            pltpu.SemaphoreType.DMA,
        ],
        compiler_params=pltpu.CompilerParams(use_tc_tiling_on_sc=False),
    )


@functools.cache
def _pm_sc(with_w):
    npar = 3 + (_F if with_w else 0)
    return pl.kernel(
        functools.partial(_pm_body, with_w),
        out_type=[
            jax.ShapeDtypeStruct((2, _N, _F), jnp.float32),
            jax.ShapeDtypeStruct((_N, _F), jnp.float32),
        ],
        mesh=_sc_mesh(),
        scratch_types=[
            _IDX_T, _IDX_T, _ROWS_T, _ROWS_T,
            _SLICE_T, _SLICE_T, _SLICE_T, _SLICE_T, _SLICE_T,
            pltpu.VMEM((npar, _F), jnp.float32),
            pltpu.VMEM((2, _F), jnp.float32),
            pltpu.VMEM((_NTILES, 2, _F), jnp.float32),
            pltpu.VMEM_SHARED((_N, _F), jnp.float32),
            pltpu.VMEM_SHARED((_N, _F), jnp.float32),
            pltpu.VMEM_SHARED((_NTILES, 2, _F), jnp.float32),
            *_SEMS4,
        ],
        compiler_params=pltpu.CompilerParams(use_tc_tiling_on_sc=False),
    )


_R = 400


def _mm1_body(x_ref, w_ref, z1_ref):
    z1_ref[...] = jnp.dot(
        x_ref[...], w_ref[...], preferred_element_type=jnp.float32
    )


def _mm1_tc(x, W1):
    nb = _N // _R
    return pl.pallas_call(
        _mm1_body,
        grid=(nb,),
        in_specs=[
            pl.BlockSpec((_R, 128), lambda i: (i, 0)),
            pl.BlockSpec((128, _F), lambda i: (0, 0)),
        ],
        out_specs=pl.BlockSpec((_R, _F), lambda i: (i, 0)),
        out_shape=jax.ShapeDtypeStruct((_N, _F), jnp.float32),
    )(x, W1)


def _post_body(sp_ref, as_ref, dinv_ref, w_ref, b_ref, out_ref):
    p = dinv_ref[...] * (sp_ref[0] + sp_ref[1] + as_ref[...])
    out_ref[...] = (
        jnp.dot(p, w_ref[...], preferred_element_type=jnp.float32) + b_ref[...]
    )


def _post_tc(sp, a2s, dinv_b, W3, b3):
    nb = _N // _R
    return pl.pallas_call(
        _post_body,
        grid=(nb,),
        in_specs=[
            pl.BlockSpec((2, _R, _F), lambda i: (0, i, 0)),
            pl.BlockSpec((_R, _F), lambda i: (i, 0)),
            pl.BlockSpec((_R, _F), lambda i: (i, 0)),
            pl.BlockSpec((_F, 128), lambda i: (0, 0)),
            pl.BlockSpec((1, 128), lambda i: (0, 0)),
        ],
        out_specs=pl.BlockSpec((_R, 128), lambda i: (i, 0)),
        out_shape=jax.ShapeDtypeStruct((_N, 128), jnp.float32),
    )(sp, a2s, dinv_b, W3, b3.reshape(1, 128))


def kernel(x, edge_index, W1, b1, g1, be1, W2, b2, g2, be2, W3, b3):
    src = edge_index[0].reshape(_E // _C, _C)
    dst = edge_index[1].reshape(_E // _C, _C)
    par1 = jnp.concatenate([b1[None, :], g1[None, :], be1[None, :], W2], axis=0)
    par2 = jnp.concatenate([b2[None, :], g2[None, :], be2[None, :]], axis=0)

    degp = _deg_sc()(dst)
    z1 = _mm1_tc(x, W1)
    s1p, z1s, dinv_b = _p1_sc()(src, dst, degp, z1)
    s2p, z2s = _pm_sc(True)(src, dst, s1p, z1s, dinv_b, par1)
    s3p, a2s = _pm_sc(False)(src, dst, s2p, z2s, dinv_b, par2)
    return _post_tc(s3p, a2s, dinv_b, W3, b3)


# speedup vs baseline: 1.1086x; 1.0490x over previous
import functools

import jax
import jax.numpy as jnp
from jax import lax
from jax.experimental import pallas as pl
from jax.experimental.pallas import tpu as pltpu
from jax.experimental.pallas import tpu_sc as plsc

_N = 10000
_E = 320000
_F = 16
_NTILES = 16
_NW = 32
_EPT = _E // _NW
_C = 80
_M = _EPT // _C
_K = 5
_NB = _M // _K
_RPT = _N // _NTILES


@functools.cache
def _sc_mesh():
    return plsc.VectorSubcoreMesh(
        core_axis_name="c", subcore_axis_name="s", num_cores=2, num_subcores=_NTILES
    )


def _rsqrt_nr(x, iters=3):
    i = lax.bitcast_convert_type(x, jnp.int32)
    y = lax.bitcast_convert_type(jnp.int32(0x5F3759DF) - (i >> 1), jnp.float32)
    for _ in range(iters):
        y = y * (1.5 - 0.5 * x * y * y)
    return y


def _zero_slice(tmp, acc_sh, s):
    def zloop(j, carry):
        tmp[j] = jnp.zeros((_F,), jnp.float32)
        return carry
    lax.fori_loop(0, _RPT, zloop, 0)
    pltpu.sync_copy(tmp, acc_sh.at[pl.ds(s * _RPT, _RPT)])


def _writeback(acc_sh, out_hbm, c, s):
    pltpu.sync_copy(
        acc_sh.at[pl.ds(s * _RPT, _RPT)],
        out_hbm.at[c, pl.ds(s * _RPT, _RPT)],
    )


def _load_idx(src_hbm, dst_hbm, src_idx, dst_idx, wid):
    pltpu.sync_copy(src_hbm.at[pl.ds(wid * _M, _M)], src_idx)
    pltpu.sync_copy(dst_hbm.at[pl.ds(wid * _M, _M)], dst_idx)


def _prop_phase(z_sh, dummy_hbm, acc_sh, src_idx, dst_idx,
                rows_a, rows_b, sga, sgb, ssa, ssb):
    def fire_gathers(j, buf, sem):
        for i in range(_K):
            pltpu.async_copy(
                z_sh.at[src_idx.at[j * _K + i]], buf.at[pl.ds(i * _C, _C)], sem
            )

    def fire_scatters(j, buf, sem):
        for i in range(_K):
            pltpu.async_copy(
                buf.at[pl.ds(i * _C, _C)], acc_sh.at[dst_idx.at[j * _K + i]],
                sem, add=True,
            )

    def drain(sem, buf):
        pltpu.make_async_copy(dummy_hbm.at[pl.ds(0, _K * _C)], buf, sem).wait()

    def body(j, carry):
        even = lax.rem(j, 2) == 0

        @pl.when(even)
        def _():
            @pl.when(j > 1)
            def _():
                drain(ssa, rows_a)
            fire_gathers(j, rows_a, sga)

            @pl.when(j > 0)
            def _():
                drain(sgb, rows_b)
                fire_scatters(j - 1, rows_b, ssb)

        @pl.when(jnp.logical_not(even))
        def _():
            @pl.when(j > 1)
            def _():
                drain(ssb, rows_b)
            fire_gathers(j, rows_b, sgb)
            drain(sga, rows_a)
            fire_scatters(j - 1, rows_a, ssa)

        return carry

    lax.fori_loop(0, _NB, body, 0)
    if _NB % 2 == 1:
        drain(sga, rows_a)
        fire_scatters(_NB - 1, rows_a, ssa)
        drain(ssb, rows_b)
        drain(ssa, rows_a)
    else:
        drain(sgb, rows_b)
        fire_scatters(_NB - 1, rows_b, ssb)
        drain(ssa, rows_a)
        drain(ssb, rows_b)


def _deg_body(dst_hbm, out_hbm, dst_idx, ones_v, drain_v, zbuf, acc_sh, sems):
    c = lax.axis_index("c")
    s = lax.axis_index("s")
    wid = c * _NTILES + s

    def oloop(j, carry):
        ones_v[j] = jnp.ones((_F,), jnp.float32)
        return carry
    lax.fori_loop(0, _C, oloop, 0)
    pltpu.sync_copy(dst_hbm.at[pl.ds(wid * _M, _M)], dst_idx)
    _zero_slice(zbuf, acc_sh, s)
    plsc.subcore_barrier()

    def _drain(sem):
        pltpu.make_async_copy(
            out_hbm.at[0, pl.ds(0, _K * _C)], drain_v, sem
        ).wait()

    def body(j, carry):
        @pl.when(j > 0)
        def _():
            _drain(sems)
        for i in range(_K):
            g = j * _K + i
            pltpu.async_copy(ones_v, acc_sh.at[dst_idx.at[g]], sems, add=True)
        return carry

    lax.fori_loop(0, _NB, body, 0)
    _drain(sems)

    plsc.subcore_barrier()
    _writeback(acc_sh, out_hbm, c, s)


def _p1_body(src_hbm, dst_hbm, degp_hbm, z1_hbm,
             out_hbm, zs_hbm, dinv_hbm,
             src_idx, dst_idx, rows_a, rows_b, dp0b, dp1b, zlocal,
             z_sh, acc_sh, sga, sgb, ssa, ssb, swb):
    c = lax.axis_index("c")
    s = lax.axis_index("s")
    wid = c * _NTILES + s
    r0 = s * _RPT

    pltpu.async_copy(src_hbm.at[pl.ds(wid * _M, _M)], src_idx, sgb)
    pltpu.async_copy(dst_hbm.at[pl.ds(wid * _M, _M)], dst_idx, sgb)
    pltpu.async_copy(degp_hbm.at[0, pl.ds(r0, _RPT)], dp0b, sga)
    pltpu.async_copy(degp_hbm.at[1, pl.ds(r0, _RPT)], dp1b, sga)
    pltpu.async_copy(z1_hbm.at[pl.ds(r0, _RPT)], zlocal, sga)
    pltpu.make_async_copy(degp_hbm.at[0, pl.ds(r0, _RPT)], dp0b, sga).wait()
    pltpu.make_async_copy(degp_hbm.at[1, pl.ds(r0, _RPT)], dp1b, sga).wait()
    pltpu.make_async_copy(z1_hbm.at[pl.ds(r0, _RPT)], zlocal, sga).wait()

    def drow(r, carry):
        di = _rsqrt_nr(dp0b[r] + dp1b[r] + 1.0, iters=2)
        zlocal[r] = zlocal[r] * di
        dp0b[r] = di
        return carry

    lax.fori_loop(0, _RPT, drow, 0)
    pltpu.sync_copy(zlocal, z_sh.at[pl.ds(r0, _RPT)])

    @pl.when(c == 0)
    def _():
        pltpu.async_copy(zlocal, zs_hbm.at[pl.ds(r0, _RPT)], swb)
        pltpu.async_copy(dp0b, dinv_hbm.at[pl.ds(r0, _RPT)], swb)

    _zero_slice(dp1b, acc_sh, s)
    pltpu.make_async_copy(src_hbm.at[pl.ds(wid * _M, _M)], src_idx, sgb).wait()
    pltpu.make_async_copy(dst_hbm.at[pl.ds(wid * _M, _M)], dst_idx, sgb).wait()
    plsc.subcore_barrier()
    _prop_phase(z_sh, z1_hbm, acc_sh, src_idx, dst_idx,
                rows_a, rows_b, sga, sgb, ssa, ssb)

    @pl.when(c == 0)
    def _():
        pltpu.make_async_copy(zlocal, zs_hbm.at[pl.ds(r0, _RPT)], swb).wait()
        pltpu.make_async_copy(dp0b, dinv_hbm.at[pl.ds(r0, _RPT)], swb).wait()

    plsc.subcore_barrier()
    _writeback(acc_sh, out_hbm, c, s)


def _pm_body(with_w, src_hbm, dst_hbm, sp_hbm, zs_hbm, dinv_hbm, par_hbm,
             out_hbm, znext_hbm,
             src_idx, dst_idx, rows_a, rows_b,
             sp0b, sp1b, zsb, dinvb, hb, wbuf, sbuf, stb,
             z_sh, acc_sh, stats_sh, sga, sgb, ssa, ssb):
    c = lax.axis_index("c")
    s = lax.axis_index("s")
    wid = c * _NTILES + s
    r0 = s * _RPT

    pltpu.async_copy(src_hbm.at[pl.ds(wid * _M, _M)], src_idx, sgb)
    pltpu.async_copy(dst_hbm.at[pl.ds(wid * _M, _M)], dst_idx, sgb)
    pltpu.async_copy(sp_hbm.at[0, pl.ds(r0, _RPT)], sp0b, sga)
    pltpu.async_copy(sp_hbm.at[1, pl.ds(r0, _RPT)], sp1b, sga)
    pltpu.async_copy(zs_hbm.at[pl.ds(r0, _RPT)], zsb, sga)
    pltpu.async_copy(dinv_hbm.at[pl.ds(r0, _RPT)], dinvb, sga)
    pltpu.async_copy(par_hbm, wbuf, sga)
    pltpu.make_async_copy(sp_hbm.at[0, pl.ds(r0, _RPT)], sp0b, sga).wait()
    pltpu.make_async_copy(sp_hbm.at[1, pl.ds(r0, _RPT)], sp1b, sga).wait()
    pltpu.make_async_copy(zs_hbm.at[pl.ds(r0, _RPT)], zsb, sga).wait()
    pltpu.make_async_copy(dinv_hbm.at[pl.ds(r0, _RPT)], dinvb, sga).wait()
    pltpu.make_async_copy(par_hbm, wbuf, sga).wait()

    bvec = wbuf[0]
    gvec = wbuf[1]
    bevec = wbuf[2]
    zero16 = jnp.zeros((_F,), jnp.float32)

    def arow(r, carry):
        ssum, ssq = carry
        h = dinvb[r] * (sp0b[r] + sp1b[r] + zsb[r]) + bvec
        hb[r] = h
        return (ssum + h, ssq + h * h)

    ssum, ssq = lax.fori_loop(0, _RPT, arow, (zero16, zero16))
    sbuf[0] = ssum
    sbuf[1] = ssq
    pltpu.sync_copy(sbuf, stats_sh.at[s])
    plsc.subcore_barrier()
    pltpu.sync_copy(stats_sh, stb)

    def sred(t, carry):
        return (carry[0] + stb[t, 0], carry[1] + stb[t, 1])

    tsum, tsq = lax.fori_loop(0, _NTILES, sred, (zero16, zero16))
    mean = tsum * (1.0 / _N)
    var = tsq * (1.0 / _N) - mean * mean
    scale = gvec * _rsqrt_nr(var + 1e-5)
    shift = bevec - mean * scale

    if with_w:
        wk = [wbuf[3 + k] for k in range(_F)]

    def brow(r, carry):
        a = jnp.maximum(hb[r] * scale + shift, 0.0)
        if with_w:
            z = zero16
            for k in range(_F):
                z = z + a[k] * wk[k]
        else:
            z = a
        zsb[r] = z * dinvb[r]
        return carry

    lax.fori_loop(0, _RPT, brow, 0)

    pltpu.sync_copy(zsb, z_sh.at[pl.ds(r0, _RPT)])

    @pl.when(c == 0)
    def _():
        pltpu.sync_copy(zsb, znext_hbm.at[pl.ds(r0, _RPT)])

    _zero_slice(sp0b, acc_sh, s)
    pltpu.make_async_copy(src_hbm.at[pl.ds(wid * _M, _M)], src_idx, sgb).wait()
    pltpu.make_async_copy(dst_hbm.at[pl.ds(wid * _M, _M)], dst_idx, sgb).wait()
    plsc.subcore_barrier()
    _prop_phase(z_sh, zs_hbm, acc_sh, src_idx, dst_idx,
                rows_a, rows_b, sga, sgb, ssa, ssb)
    plsc.subcore_barrier()
    _writeback(acc_sh, out_hbm, c, s)


_IDX_T = pltpu.VMEM((_M, _C), jnp.int32)
_ROWS_T = pltpu.VMEM((_K * _C, _F), jnp.float32)
_SLICE_T = pltpu.VMEM((_RPT, _F), jnp.float32)
_SEMS4 = [pltpu.SemaphoreType.DMA] * 4


@functools.cache
def _deg_sc():
    return pl.kernel(
        _deg_body,
        out_type=jax.ShapeDtypeStruct((2, _N, _F), jnp.float32),
        mesh=_sc_mesh(),
        scratch_types=[
            _IDX_T,
            pltpu.VMEM((_C, _F), jnp.float32),
            _ROWS_T,
            _SLICE_T,
            pltpu.VMEM_SHARED((_N, _F), jnp.float32),
            pltpu.SemaphoreType.DMA,
        ],
        compiler_params=pltpu.CompilerParams(use_tc_tiling_on_sc=False),
    )


@functools.cache
def _p1_sc():
    return pl.kernel(
        _p1_body,
        out_type=[
            jax.ShapeDtypeStruct((2, _N, _F), jnp.float32),
            jax.ShapeDtypeStruct((_N, _F), jnp.float32),
            jax.ShapeDtypeStruct((_N, _F), jnp.float32),
        ],
        mesh=_sc_mesh(),
        scratch_types=[
            _IDX_T, _IDX_T, _ROWS_T, _ROWS_T,
            _SLICE_T, _SLICE_T, _SLICE_T,
            pltpu.VMEM_SHARED((_N, _F), jnp.float32),
            pltpu.VMEM_SHARED((_N, _F), jnp.float32),
            *_SEMS4,
            pltpu.SemaphoreType.DMA,
        ],
        compiler_params=pltpu.CompilerParams(use_tc_tiling_on_sc=False),
    )


@functools.cache
def _pm_sc(with_w):
    npar = 3 + (_F if with_w else 0)
    return pl.kernel(
        functools.partial(_pm_body, with_w),
        out_type=[
            jax.ShapeDtypeStruct((2, _N, _F), jnp.float32),
            jax.ShapeDtypeStruct((_N, _F), jnp.float32),
        ],
        mesh=_sc_mesh(),
        scratch_types=[
            _IDX_T, _IDX_T, _ROWS_T, _ROWS_T,
            _SLICE_T, _SLICE_T, _SLICE_T, _SLICE_T, _SLICE_T,
            pltpu.VMEM((npar, _F), jnp.float32),
            pltpu.VMEM((2, _F), jnp.float32),
            pltpu.VMEM((_NTILES, 2, _F), jnp.float32),
            pltpu.VMEM_SHARED((_N, _F), jnp.float32),
            pltpu.VMEM_SHARED((_N, _F), jnp.float32),
            pltpu.VMEM_SHARED((_NTILES, 2, _F), jnp.float32),
            *_SEMS4,
        ],
        compiler_params=pltpu.CompilerParams(use_tc_tiling_on_sc=False),
    )


_R = 400


def _mm1_body(x_ref, w_ref, z1_ref):
    z1_ref[...] = jnp.dot(
        x_ref[...], w_ref[...], preferred_element_type=jnp.float32
    )


def _mm1_tc(x, W1):
    nb = _N // _R
    return pl.pallas_call(
        _mm1_body,
        grid=(nb,),
        in_specs=[
            pl.BlockSpec((_R, 128), lambda i: (i, 0)),
            pl.BlockSpec((128, _F), lambda i: (0, 0)),
        ],
        out_specs=pl.BlockSpec((_R, _F), lambda i: (i, 0)),
        out_shape=jax.ShapeDtypeStruct((_N, _F), jnp.float32),
    )(x, W1)


def _post_body(sp_ref, as_ref, dinv_ref, w_ref, b_ref, out_ref):
    p = dinv_ref[...] * (sp_ref[0] + sp_ref[1] + as_ref[...])
    out_ref[...] = (
        jnp.dot(p, w_ref[...], preferred_element_type=jnp.float32) + b_ref[...]
    )


def _post_tc(sp, a2s, dinv_b, W3, b3):
    nb = _N // _R
    return pl.pallas_call(
        _post_body,
        grid=(nb,),
        in_specs=[
            pl.BlockSpec((2, _R, _F), lambda i: (0, i, 0)),
            pl.BlockSpec((_R, _F), lambda i: (i, 0)),
            pl.BlockSpec((_R, _F), lambda i: (i, 0)),
            pl.BlockSpec((_F, 128), lambda i: (0, 0)),
            pl.BlockSpec((1, 128), lambda i: (0, 0)),
        ],
        out_specs=pl.BlockSpec((_R, 128), lambda i: (i, 0)),
        out_shape=jax.ShapeDtypeStruct((_N, 128), jnp.float32),
    )(sp, a2s, dinv_b, W3, b3.reshape(1, 128))


def kernel(x, edge_index, W1, b1, g1, be1, W2, b2, g2, be2, W3, b3):
    src = edge_index[0].reshape(_E // _C, _C)
    dst = edge_index[1].reshape(_E // _C, _C)
    par1 = jnp.concatenate([b1[None, :], g1[None, :], be1[None, :], W2], axis=0)
    par2 = jnp.concatenate([b2[None, :], g2[None, :], be2[None, :]], axis=0)

    degp = _deg_sc()(dst)
    z1 = _mm1_tc(x, W1)
    s1p, z1s, dinv_b = _p1_sc()(src, dst, degp, z1)
    s2p, z2s = _pm_sc(True)(src, dst, s1p, z1s, dinv_b, par1)
    s3p, a2s = _pm_sc(False)(src, dst, s2p, z2s, dinv_b, par2)
    return _post_tc(s3p, a2s, dinv_b, W3, b3)


# speedup vs baseline: 1.1089x; 1.0003x over previous
import functools

import jax
import jax.numpy as jnp
from jax import lax
from jax.experimental import pallas as pl
from jax.experimental.pallas import tpu as pltpu
from jax.experimental.pallas import tpu_sc as plsc

_N = 10000
_E = 320000
_F = 16
_NTILES = 16
_NW = 32
_EPT = _E // _NW
_C = 80
_M = _EPT // _C
_K = 5
_NB = _M // _K
_RPT = _N // _NTILES


@functools.cache
def _sc_mesh():
    return plsc.VectorSubcoreMesh(
        core_axis_name="c", subcore_axis_name="s", num_cores=2, num_subcores=_NTILES
    )


def _rsqrt_nr(x, iters=3):
    i = lax.bitcast_convert_type(x, jnp.int32)
    y = lax.bitcast_convert_type(jnp.int32(0x5F3759DF) - (i >> 1), jnp.float32)
    for _ in range(iters):
        y = y * (1.5 - 0.5 * x * y * y)
    return y


def _zero_slice(tmp, acc_sh, s):
    def zloop(j, carry):
        tmp[j] = jnp.zeros((_F,), jnp.float32)
        return carry
    lax.fori_loop(0, _RPT, zloop, 0)
    pltpu.sync_copy(tmp, acc_sh.at[pl.ds(s * _RPT, _RPT)])


def _writeback(acc_sh, out_hbm, c, s):
    pltpu.sync_copy(
        acc_sh.at[pl.ds(s * _RPT, _RPT)],
        out_hbm.at[c, pl.ds(s * _RPT, _RPT)],
    )


def _load_idx(src_hbm, dst_hbm, src_idx, dst_idx, wid):
    pltpu.sync_copy(src_hbm.at[pl.ds(wid * _M, _M)], src_idx)
    pltpu.sync_copy(dst_hbm.at[pl.ds(wid * _M, _M)], dst_idx)


def _prop_phase(z_sh, dummy_hbm, acc_sh, src_idx, dst_idx,
                rows_a, rows_b, sga, sgb, ssa, ssb):
    def fire_gathers(j, buf, sem):
        for i in range(_K):
            pltpu.async_copy(
                z_sh.at[src_idx.at[j * _K + i]], buf.at[pl.ds(i * _C, _C)], sem
            )

    def fire_scatters(j, buf, sem):
        for i in range(_K):
            pltpu.async_copy(
                buf.at[pl.ds(i * _C, _C)], acc_sh.at[dst_idx.at[j * _K + i]],
                sem, add=True,
            )

    def drain(sem, buf):
        pltpu.make_async_copy(dummy_hbm.at[pl.ds(0, _K * _C)], buf, sem).wait()

    def body(j, carry):
        even = lax.rem(j, 2) == 0

        @pl.when(even)
        def _():
            @pl.when(j > 1)
            def _():
                drain(ssa, rows_a)
            fire_gathers(j, rows_a, sga)

            @pl.when(j > 0)
            def _():
                drain(sgb, rows_b)
                fire_scatters(j - 1, rows_b, ssb)

        @pl.when(jnp.logical_not(even))
        def _():
            @pl.when(j > 1)
            def _():
                drain(ssb, rows_b)
            fire_gathers(j, rows_b, sgb)
            drain(sga, rows_a)
            fire_scatters(j - 1, rows_a, ssa)

        return carry

    lax.fori_loop(0, _NB, body, 0)
    if _NB % 2 == 1:
        drain(sga, rows_a)
        fire_scatters(_NB - 1, rows_a, ssa)
        drain(ssb, rows_b)
        drain(ssa, rows_a)
    else:
        drain(sgb, rows_b)
        fire_scatters(_NB - 1, rows_b, ssb)
        drain(ssa, rows_a)
        drain(ssb, rows_b)


def _deg_body(dst_hbm, out_hbm, dst_idx, ones_v, drain_v, zbuf, acc_sh, sems):
    c = lax.axis_index("c")
    s = lax.axis_index("s")
    wid = c * _NTILES + s

    pltpu.async_copy(dst_hbm.at[pl.ds(wid * _M, _M)], dst_idx, sems)

    def oloop(j, carry):
        ones_v[j] = jnp.ones((_F,), jnp.float32)
        return carry
    lax.fori_loop(0, _C, oloop, 0)
    _zero_slice(zbuf, acc_sh, s)
    pltpu.make_async_copy(dst_hbm.at[pl.ds(wid * _M, _M)], dst_idx, sems).wait()
    plsc.subcore_barrier()

    def _drain(sem):
        pltpu.make_async_copy(
            out_hbm.at[0, pl.ds(0, _K * _C)], drain_v, sem
        ).wait()

    def body(j, carry):
        @pl.when(j > 0)
        def _():
            _drain(sems)
        for i in range(_K):
            g = j * _K + i
            pltpu.async_copy(ones_v, acc_sh.at[dst_idx.at[g]], sems, add=True)
        return carry

    lax.fori_loop(0, _NB, body, 0)
    _drain(sems)

    plsc.subcore_barrier()
    _writeback(acc_sh, out_hbm, c, s)


def _p1_body(src_hbm, dst_hbm, degp_hbm, z1_hbm,
             out_hbm, zs_hbm, dinv_hbm,
             src_idx, dst_idx, rows_a, rows_b, dp0b, dp1b, zlocal,
             z_sh, acc_sh, sga, sgb, ssa, ssb, swb):
    c = lax.axis_index("c")
    s = lax.axis_index("s")
    wid = c * _NTILES + s
    r0 = s * _RPT

    pltpu.async_copy(src_hbm.at[pl.ds(wid * _M, _M)], src_idx, sgb)
    pltpu.async_copy(dst_hbm.at[pl.ds(wid * _M, _M)], dst_idx, sgb)
    pltpu.async_copy(degp_hbm.at[0, pl.ds(r0, _RPT)], dp0b, sga)
    pltpu.async_copy(degp_hbm.at[1, pl.ds(r0, _RPT)], dp1b, sga)
    pltpu.async_copy(z1_hbm.at[pl.ds(r0, _RPT)], zlocal, sga)
    pltpu.make_async_copy(degp_hbm.at[0, pl.ds(r0, _RPT)], dp0b, sga).wait()
    pltpu.make_async_copy(degp_hbm.at[1, pl.ds(r0, _RPT)], dp1b, sga).wait()
    pltpu.make_async_copy(z1_hbm.at[pl.ds(r0, _RPT)], zlocal, sga).wait()

    def drow(r, carry):
        di = _rsqrt_nr(dp0b[r] + dp1b[r] + 1.0, iters=2)
        zlocal[r] = zlocal[r] * di
        dp0b[r] = di
        return carry

    lax.fori_loop(0, _RPT, drow, 0)
    pltpu.sync_copy(zlocal, z_sh.at[pl.ds(r0, _RPT)])

    @pl.when(c == 0)
    def _():
        pltpu.async_copy(zlocal, zs_hbm.at[pl.ds(r0, _RPT)], swb)
        pltpu.async_copy(dp0b, dinv_hbm.at[pl.ds(r0, _RPT)], swb)

    _zero_slice(dp1b, acc_sh, s)
    pltpu.make_async_copy(src_hbm.at[pl.ds(wid * _M, _M)], src_idx, sgb).wait()
    pltpu.make_async_copy(dst_hbm.at[pl.ds(wid * _M, _M)], dst_idx, sgb).wait()
    plsc.subcore_barrier()
    _prop_phase(z_sh, z1_hbm, acc_sh, src_idx, dst_idx,
                rows_a, rows_b, sga, sgb, ssa, ssb)

    @pl.when(c == 0)
    def _():
        pltpu.make_async_copy(zlocal, zs_hbm.at[pl.ds(r0, _RPT)], swb).wait()
        pltpu.make_async_copy(dp0b, dinv_hbm.at[pl.ds(r0, _RPT)], swb).wait()

    plsc.subcore_barrier()
    _writeback(acc_sh, out_hbm, c, s)


def _pm_body(with_w, src_hbm, dst_hbm, sp_hbm, zs_hbm, dinv_hbm, par_hbm,
             out_hbm, znext_hbm,
             src_idx, dst_idx, rows_a, rows_b,
             sp0b, sp1b, zsb, dinvb, hb, wbuf, sbuf, stb,
             z_sh, acc_sh, stats_sh, sga, sgb, ssa, ssb):
    c = lax.axis_index("c")
    s = lax.axis_index("s")
    wid = c * _NTILES + s
    r0 = s * _RPT

    pltpu.async_copy(src_hbm.at[pl.ds(wid * _M, _M)], src_idx, sgb)
    pltpu.async_copy(dst_hbm.at[pl.ds(wid * _M, _M)], dst_idx, sgb)
    pltpu.async_copy(sp_hbm.at[0, pl.ds(r0, _RPT)], sp0b, sga)
    pltpu.async_copy(sp_hbm.at[1, pl.ds(r0, _RPT)], sp1b, sga)
    pltpu.async_copy(zs_hbm.at[pl.ds(r0, _RPT)], zsb, sga)
    pltpu.async_copy(dinv_hbm.at[pl.ds(r0, _RPT)], dinvb, sga)
    pltpu.async_copy(par_hbm, wbuf, sga)
    pltpu.make_async_copy(sp_hbm.at[0, pl.ds(r0, _RPT)], sp0b, sga).wait()
    pltpu.make_async_copy(sp_hbm.at[1, pl.ds(r0, _RPT)], sp1b, sga).wait()
    pltpu.make_async_copy(zs_hbm.at[pl.ds(r0, _RPT)], zsb, sga).wait()
    pltpu.make_async_copy(dinv_hbm.at[pl.ds(r0, _RPT)], dinvb, sga).wait()
    pltpu.make_async_copy(par_hbm, wbuf, sga).wait()

    bvec = wbuf[0]
    gvec = wbuf[1]
    bevec = wbuf[2]
    zero16 = jnp.zeros((_F,), jnp.float32)

    def arow(r, carry):
        ssum, ssq = carry
        h = dinvb[r] * (sp0b[r] + sp1b[r] + zsb[r]) + bvec
        hb[r] = h
        return (ssum + h, ssq + h * h)

    ssum, ssq = lax.fori_loop(0, _RPT, arow, (zero16, zero16))
    sbuf[0] = ssum
    sbuf[1] = ssq
    pltpu.sync_copy(sbuf, stats_sh.at[s])
    plsc.subcore_barrier()
    pltpu.sync_copy(stats_sh, stb)

    def sred(t, carry):
        return (carry[0] + stb[t, 0], carry[1] + stb[t, 1])

    tsum, tsq = lax.fori_loop(0, _NTILES, sred, (zero16, zero16))
    mean = tsum * (1.0 / _N)
    var = tsq * (1.0 / _N) - mean * mean
    scale = gvec * _rsqrt_nr(var + 1e-5)
    shift = bevec - mean * scale

    if with_w:
        wk = [wbuf[3 + k] for k in range(_F)]

    def brow(r, carry):
        a = jnp.maximum(hb[r] * scale + shift, 0.0)
        if with_w:
            z = zero16
            for k in range(_F):
                z = z + a[k] * wk[k]
        else:
            z = a
        zsb[r] = z * dinvb[r]
        return carry

    lax.fori_loop(0, _RPT, brow, 0)

    pltpu.sync_copy(zsb, z_sh.at[pl.ds(r0, _RPT)])

    @pl.when(c == 0)
    def _():
        pltpu.sync_copy(zsb, znext_hbm.at[pl.ds(r0, _RPT)])

    _zero_slice(sp0b, acc_sh, s)
    pltpu.make_async_copy(src_hbm.at[pl.ds(wid * _M, _M)], src_idx, sgb).wait()
    pltpu.make_async_copy(dst_hbm.at[pl.ds(wid * _M, _M)], dst_idx, sgb).wait()
    plsc.subcore_barrier()
    _prop_phase(z_sh, zs_hbm, acc_sh, src_idx, dst_idx,
                rows_a, rows_b, sga, sgb, ssa, ssb)
    plsc.subcore_barrier()
    _writeback(acc_sh, out_hbm, c, s)


_IDX_T = pltpu.VMEM((_M, _C), jnp.int32)
_ROWS_T = pltpu.VMEM((_K * _C, _F), jnp.float32)
_SLICE_T = pltpu.VMEM((_RPT, _F), jnp.float32)
_SEMS4 = [pltpu.SemaphoreType.DMA] * 4


@functools.cache
def _deg_sc():
    return pl.kernel(
        _deg_body,
        out_type=jax.ShapeDtypeStruct((2, _N, _F), jnp.float32),
        mesh=_sc_mesh(),
        scratch_types=[
            _IDX_T,
            pltpu.VMEM((_C, _F), jnp.float32),
            _ROWS_T,
            _SLICE_T,
            pltpu.VMEM_SHARED((_N, _F), jnp.float32),
            pltpu.SemaphoreType.DMA,
        ],
        compiler_params=pltpu.CompilerParams(use_tc_tiling_on_sc=False),
    )


@functools.cache
def _p1_sc():
    return pl.kernel(
        _p1_body,
        out_type=[
            jax.ShapeDtypeStruct((2, _N, _F), jnp.float32),
            jax.ShapeDtypeStruct((_N, _F), jnp.float32),
            jax.ShapeDtypeStruct((_N, _F), jnp.float32),
        ],
        mesh=_sc_mesh(),
        scratch_types=[
            _IDX_T, _IDX_T, _ROWS_T, _ROWS_T,
            _SLICE_T, _SLICE_T, _SLICE_T,
            pltpu.VMEM_SHARED((_N, _F), jnp.float32),
            pltpu.VMEM_SHARED((_N, _F), jnp.float32),
            *_SEMS4,
            pltpu.SemaphoreType.DMA,
        ],
        compiler_params=pltpu.CompilerParams(use_tc_tiling_on_sc=False),
    )


@functools.cache
def _pm_sc(with_w):
    npar = 3 + (_F if with_w else 0)
    return pl.kernel(
        functools.partial(_pm_body, with_w),
        out_type=[
            jax.ShapeDtypeStruct((2, _N, _F), jnp.float32),
            jax.ShapeDtypeStruct((_N, _F), jnp.float32),
        ],
        mesh=_sc_mesh(),
        scratch_types=[
            _IDX_T, _IDX_T, _ROWS_T, _ROWS_T,
            _SLICE_T, _SLICE_T, _SLICE_T, _SLICE_T, _SLICE_T,
            pltpu.VMEM((npar, _F), jnp.float32),
            pltpu.VMEM((2, _F), jnp.float32),
            pltpu.VMEM((_NTILES, 2, _F), jnp.float32),
            pltpu.VMEM_SHARED((_N, _F), jnp.float32),
            pltpu.VMEM_SHARED((_N, _F), jnp.float32),
            pltpu.VMEM_SHARED((_NTILES, 2, _F), jnp.float32),
            *_SEMS4,
        ],
        compiler_params=pltpu.CompilerParams(use_tc_tiling_on_sc=False),
    )


_R = 400


def _mm1_body(x_ref, w_ref, z1_ref):
    z1_ref[...] = jnp.dot(
        x_ref[...], w_ref[...], preferred_element_type=jnp.float32
    )


def _mm1_tc(x, W1):
    nb = _N // _R
    return pl.pallas_call(
        _mm1_body,
        grid=(nb,),
        in_specs=[
            pl.BlockSpec((_R, 128), lambda i: (i, 0)),
            pl.BlockSpec((128, _F), lambda i: (0, 0)),
        ],
        out_specs=pl.BlockSpec((_R, _F), lambda i: (i, 0)),
        out_shape=jax.ShapeDtypeStruct((_N, _F), jnp.float32),
    )(x, W1)


def _post_body(sp_ref, as_ref, dinv_ref, w_ref, b_ref, out_ref):
    p = dinv_ref[...] * (sp_ref[0] + sp_ref[1] + as_ref[...])
    out_ref[...] = (
        jnp.dot(p, w_ref[...], preferred_element_type=jnp.float32) + b_ref[...]
    )


def _post_tc(sp, a2s, dinv_b, W3, b3):
    nb = _N // _R
    return pl.pallas_call(
        _post_body,
        grid=(nb,),
        in_specs=[
            pl.BlockSpec((2, _R, _F), lambda i: (0, i, 0)),
            pl.BlockSpec((_R, _F), lambda i: (i, 0)),
            pl.BlockSpec((_R, _F), lambda i: (i, 0)),
            pl.BlockSpec((_F, 128), lambda i: (0, 0)),
            pl.BlockSpec((1, 128), lambda i: (0, 0)),
        ],
        out_specs=pl.BlockSpec((_R, 128), lambda i: (i, 0)),
        out_shape=jax.ShapeDtypeStruct((_N, 128), jnp.float32),
    )(sp, a2s, dinv_b, W3, b3.reshape(1, 128))


def kernel(x, edge_index, W1, b1, g1, be1, W2, b2, g2, be2, W3, b3):
    src = edge_index[0].reshape(_E // _C, _C)
    dst = edge_index[1].reshape(_E // _C, _C)
    par1 = jnp.concatenate([b1[None, :], g1[None, :], be1[None, :], W2], axis=0)
    par2 = jnp.concatenate([b2[None, :], g2[None, :], be2[None, :]], axis=0)

    degp = _deg_sc()(dst)
    z1 = _mm1_tc(x, W1)
    s1p, z1s, dinv_b = _p1_sc()(src, dst, degp, z1)
    s2p, z2s = _pm_sc(True)(src, dst, s1p, z1s, dinv_b, par1)
    s3p, a2s = _pm_sc(False)(src, dst, s2p, z2s, dinv_b, par2)
    return _post_tc(s3p, a2s, dinv_b, W3, b3)
